# baseline (speedup 1.0000x reference)
import functools

import jax
import jax.numpy as jnp
from jax import lax
from jax.experimental import pallas as pl
from jax.experimental.pallas import tpu as pltpu
from jax.experimental.pallas import tpu_sc as plsc

NUM_ITEM = 1000000
HIDDEN = 64
BATCH = 16384
HIST = 20

_L = 16
_NC = 2
_NS = 16
_NW = _NC * _NS
_B = BATCH * HIST
_BPW = _B // _NW
_CH = 128
_NCHUNK = _BPW // _CH


def _soft_threshold_chunk(e_v, s_v):
    def row_body(r, carry):
        for j in range(HIDDEN // _L):
            sl = pl.ds(j * _L, _L)
            v = e_v[r, sl]
            t = s_v[r, sl]
            sig = 1.0 / (1.0 + jnp.exp(-t))
            e_v[r, sl] = jnp.sign(v) * jnp.maximum(jnp.abs(v) - sig, 0.0)
        return carry

    lax.fori_loop(0, _CH, row_body, 0, unroll=False)


@functools.partial(
    pl.kernel,
    out_type=jax.ShapeDtypeStruct((_B, HIDDEN), jnp.float32),
    mesh=plsc.VectorSubcoreMesh(core_axis_name="c", subcore_axis_name="s"),
    compiler_params=pltpu.CompilerParams(use_tc_tiling_on_sc=False),
    scratch_types=[
        pltpu.VMEM((_CH,), jnp.int32),
        pltpu.VMEM((_CH, HIDDEN), jnp.float32),
        pltpu.VMEM((_CH, HIDDEN), jnp.float32),
        pltpu.SemaphoreType.DMA,
        pltpu.SemaphoreType.DMA,
    ],
)
def _sc_lookup(idx_hbm, emb_hbm, s_hbm, out_hbm, idx_v, e_v, s_v, sem_e, sem_s):
    wid = lax.axis_index("s") * _NC + lax.axis_index("c")
    base = wid * _BPW

    def chunk_body(c, carry):
        off = base + c * _CH
        pltpu.sync_copy(idx_hbm.at[pl.ds(off, _CH)], idx_v)
        cp_e = pltpu.async_copy(emb_hbm.at[idx_v], e_v, sem_e)
        cp_s = pltpu.async_copy(s_hbm.at[idx_v], s_v, sem_s)
        cp_e.wait()
        cp_s.wait()
        _soft_threshold_chunk(e_v, s_v)
        pltpu.sync_copy(e_v, out_hbm.at[pl.ds(off, _CH)])
        return carry

    lax.fori_loop(0, _NCHUNK, chunk_body, 0, unroll=False)


def kernel(x, emb_weight, s):
    idx = x.reshape(-1).astype(jnp.int32)
    out = _sc_lookup(idx, emb_weight, s)
    return out.reshape(BATCH, HIST, HIDDEN)
